# baseline (speedup 1.0000x reference)
import jax
import jax.numpy as jnp
from jax import lax
from jax.experimental import pallas as pl
from jax.experimental.pallas import tpu as pltpu
from jax.experimental.pallas import tpu_sc as plsc

N = 100000
EMBED_DIM = 16
OUT_DIM = 2 * EMBED_DIM
PACK = 4
PACK_DIM = PACK * OUT_DIM
NP = N // PACK
NC, NS = 2, 16
NW = NC * NS
PROWS_W = 784
PROWS_LAST = NP - (NW - 1) * PROWS_W
ROWS_W = PROWS_W * PACK
ROWS_LAST = PROWS_LAST * PACK
L = 16


def _sc_body(nt_hbm, niv_hbm, tab_hbm, wb_hbm, out_hbm,
             nt_v, niv_v, tab_v, wb_v, out_v):
    c = lax.axis_index("c")
    s = lax.axis_index("s")
    wid = s * NC + c
    is_last = wid == NW - 1
    base = wid * ROWS_W

    pltpu.sync_copy(tab_hbm, tab_v)
    pltpu.sync_copy(wb_hbm, wb_v)

    @pl.when(jnp.logical_not(is_last))
    def _():
        pltpu.sync_copy(nt_hbm.at[pl.ds(base, ROWS_W)], nt_v)
        pltpu.sync_copy(niv_hbm.at[pl.ds(base, ROWS_W)], niv_v)

    @pl.when(is_last)
    def _():
        pltpu.sync_copy(nt_hbm.at[pl.ds(base, ROWS_LAST)],
                        nt_v.at[pl.ds(0, ROWS_LAST)])
        pltpu.sync_copy(niv_hbm.at[pl.ds(base, ROWS_LAST)],
                        niv_v.at[pl.ds(0, ROWS_LAST)])

    wv = wb_v[pl.ds(0, L)]
    bv = wb_v[pl.ds(L, L)]
    lane = lax.iota(jnp.int32, L)

    nblk = jnp.where(is_last, ROWS_LAST // L, ROWS_W // L)

    def blk_body(i, carry):
        b16 = i * L
        nt16 = nt_v[pl.ds(b16, L)]
        niv16 = niv_v[pl.ds(b16, L)]
        for r in range(L):
            rr = jnp.full((L,), r, dtype=jnp.int32)
            ntb = jnp.take_along_axis(nt16, rr, axis=0)
            nivb = jnp.take_along_axis(niv16, rr, axis=0)
            emb = plsc.load_gather(tab_v, [ntb * EMBED_DIM + lane])
            p = 4 * i + (r // 4)
            off = (r % 4) * OUT_DIM
            out_v[p, pl.ds(off, L)] = emb
            out_v[p, pl.ds(off + L, L)] = nivb * wv + bv
        return carry

    lax.fori_loop(0, nblk, blk_body, 0)

    pbase = wid * PROWS_W

    @pl.when(jnp.logical_not(is_last))
    def _():
        pltpu.sync_copy(out_v, out_hbm.at[pl.ds(pbase, PROWS_W), :])

    @pl.when(is_last)
    def _():
        pltpu.sync_copy(out_v.at[pl.ds(0, PROWS_LAST), :],
                        out_hbm.at[pl.ds(pbase, PROWS_LAST), :])


@jax.jit
def _sc_call(nt, niv, tab, wb):
    mesh = plsc.VectorSubcoreMesh(
        core_axis_name="c", subcore_axis_name="s",
        num_cores=NC, num_subcores=NS,
    )
    f = pl.kernel(
        _sc_body,
        out_type=jax.ShapeDtypeStruct((NP, PACK_DIM), jnp.float32),
        mesh=mesh,
        compiler_params=pltpu.CompilerParams(needs_layout_passes=False),
        scratch_types=[
            pltpu.VMEM((ROWS_W,), jnp.int32),
            pltpu.VMEM((ROWS_W,), jnp.float32),
            pltpu.VMEM((3 * EMBED_DIM,), jnp.float32),
            pltpu.VMEM((2 * EMBED_DIM,), jnp.float32),
            pltpu.VMEM((PROWS_W, PACK_DIM), jnp.float32),
        ],
    )
    return f(nt, niv, tab, wb)


def kernel(node_type, num_inverted_predecessors, embed_table, W, b):
    nt = node_type.astype(jnp.int32)
    wb = jnp.concatenate([W.T, b[None, :]], axis=0)
    out_p = _sc_call(nt, num_inverted_predecessors,
                     embed_table.reshape(-1), wb.reshape(-1))
    return out_p.reshape(N, OUT_DIM)


# speedup vs baseline: 1.5137x; 1.5137x over previous
import jax
import jax.numpy as jnp
from jax import lax
from jax.experimental import pallas as pl
from jax.experimental.pallas import tpu as pltpu
from jax.experimental.pallas import tpu_sc as plsc

N = 100000
EMBED_DIM = 16
OUT_DIM = 2 * EMBED_DIM
NG = N // 8
NC, NS = 2, 16
NW = NC * NS
ROWS_W = 3200
ROWS_LAST = N - (NW - 1) * ROWS_W
SUB = 400
GSUB = SUB // 8
NSUB = ROWS_W // SUB
NSUB_LAST = ROWS_LAST // SUB
L = 16


def _sc_body(nt_hbm, niv_hbm, tab_hbm, wb_hbm, out_hbm,
             nt_v, niv_v, tab_v, wb_v, buf0, buf1, sem):
    c = lax.axis_index("c")
    s = lax.axis_index("s")
    wid = s * NC + c
    is_last = wid == NW - 1
    base = wid * ROWS_W

    pltpu.sync_copy(tab_hbm, tab_v)
    pltpu.sync_copy(wb_hbm, wb_v)

    @pl.when(jnp.logical_not(is_last))
    def _():
        pltpu.sync_copy(nt_hbm.at[pl.ds(base, ROWS_W)], nt_v)
        pltpu.sync_copy(niv_hbm.at[pl.ds(base, ROWS_W)], niv_v)

    @pl.when(is_last)
    def _():
        pltpu.sync_copy(nt_hbm.at[pl.ds(base, ROWS_LAST)],
                        nt_v.at[pl.ds(0, ROWS_LAST)])
        pltpu.sync_copy(niv_hbm.at[pl.ds(base, ROWS_LAST)],
                        niv_v.at[pl.ds(0, ROWS_LAST)])

    wv = wb_v[pl.ds(0, L)]
    bv = wb_v[pl.ds(L, L)]
    lane = lax.iota(jnp.int32, L)
    gbase = wid * (ROWS_W // 8)

    def compute_chunk(k, buf):
        def blk_body(i, carry):
            b16 = k * SUB + i * L
            nt16 = nt_v[pl.ds(b16, L)]
            niv16 = niv_v[pl.ds(b16, L)]
            for r in range(L):
                rr = jnp.full((L,), r, dtype=jnp.int32)
                ntb = jnp.take_along_axis(nt16, rr, axis=0)
                nivb = jnp.take_along_axis(niv16, rr, axis=0)
                emb = plsc.load_gather(tab_v, [ntb * EMBED_DIM + lane])
                g = 2 * i + (r // 8)
                sl = r % 8
                buf[g, sl, pl.ds(0, L)] = emb
                buf[g, sl, pl.ds(L, L)] = nivb * wv + bv
            return carry

        lax.fori_loop(0, SUB // L, blk_body, 0)

    def dma(k, buf):
        dst = out_hbm.at[pl.ds(gbase + k * GSUB, GSUB), :, :]
        return pltpu.make_async_copy(buf, dst, sem)

    npair = jnp.where(is_last, NSUB_LAST // 2, NSUB // 2)

    def pair_body(j, carry):
        k0 = 2 * j
        compute_chunk(k0, buf0)

        @pl.when(j > 0)
        def _():
            dma(k0, buf0).wait()

        dma(k0, buf0).start()
        compute_chunk(k0 + 1, buf1)
        dma(k0, buf0).wait()
        dma(k0 + 1, buf1).start()
        return carry

    lax.fori_loop(0, npair, pair_body, 0)
    dma(0, buf1).wait()


@jax.jit
def _sc_call(nt, niv, tab, wb):
    mesh = plsc.VectorSubcoreMesh(
        core_axis_name="c", subcore_axis_name="s",
        num_cores=NC, num_subcores=NS,
    )
    f = pl.kernel(
        _sc_body,
        out_type=jax.ShapeDtypeStruct((NG, 8, OUT_DIM), jnp.float32),
        mesh=mesh,
        compiler_params=pltpu.CompilerParams(needs_layout_passes=False),
        scratch_types=[
            pltpu.VMEM((ROWS_W,), jnp.int32),
            pltpu.VMEM((ROWS_W,), jnp.float32),
            pltpu.VMEM((3 * EMBED_DIM,), jnp.float32),
            pltpu.VMEM((2 * EMBED_DIM,), jnp.float32),
            pltpu.VMEM((GSUB, 8, OUT_DIM), jnp.float32),
            pltpu.VMEM((GSUB, 8, OUT_DIM), jnp.float32),
            pltpu.SemaphoreType.DMA,
        ],
    )
    return f(nt, niv, tab, wb)


def kernel(node_type, num_inverted_predecessors, embed_table, W, b):
    nt = node_type.astype(jnp.int32)
    wb = jnp.concatenate([W.T, b[None, :]], axis=0)
    out_g = _sc_call(nt, num_inverted_predecessors,
                     embed_table.reshape(-1), wb.reshape(-1))
    return out_g.reshape(N, OUT_DIM)


# speedup vs baseline: 1.5501x; 1.0240x over previous
import jax
import jax.numpy as jnp
from jax import lax
from jax.experimental import pallas as pl
from jax.experimental.pallas import tpu as pltpu
from jax.experimental.pallas import tpu_sc as plsc

N = 100000
EMBED_DIM = 16
OUT_DIM = 2 * EMBED_DIM
NG = N // 8
NC, NS = 2, 16
NW = NC * NS
ROWS_W = 3200
ROWS_LAST = N - (NW - 1) * ROWS_W
SUB = 400
GSUB = SUB // 8
NSUB = ROWS_W // SUB
NSUB_LAST = ROWS_LAST // SUB
L = 16


def _sc_body(nt_hbm, niv_hbm, tab_hbm, wb_hbm, out_hbm,
             nt_v, niv_v, tab_v, wb_v, buf, sem):
    c = lax.axis_index("c")
    s = lax.axis_index("s")
    wid = s * NC + c
    is_last = wid == NW - 1
    base = wid * ROWS_W

    pltpu.sync_copy(tab_hbm, tab_v)
    pltpu.sync_copy(wb_hbm, wb_v)

    @pl.when(jnp.logical_not(is_last))
    def _():
        pltpu.sync_copy(nt_hbm.at[pl.ds(base, ROWS_W)], nt_v)
        pltpu.sync_copy(niv_hbm.at[pl.ds(base, ROWS_W)], niv_v)

    @pl.when(is_last)
    def _():
        pltpu.sync_copy(nt_hbm.at[pl.ds(base, ROWS_LAST)],
                        nt_v.at[pl.ds(0, ROWS_LAST)])
        pltpu.sync_copy(niv_hbm.at[pl.ds(base, ROWS_LAST)],
                        niv_v.at[pl.ds(0, ROWS_LAST)])

    wv = wb_v[pl.ds(0, L)]
    bv = wb_v[pl.ds(L, L)]
    lane = lax.iota(jnp.int32, L)
    gbase = wid * (ROWS_W // 8)

    def dma(k, par):
        dst = out_hbm.at[pl.ds(gbase + k * GSUB, GSUB), :, :]
        return pltpu.make_async_copy(buf.at[par], dst, sem)

    nsub = jnp.where(is_last, NSUB_LAST, NSUB)

    def chunk_body(k, carry):
        par = lax.rem(k, 2)

        @plsc.parallel_loop(0, SUB // L)
        def _blocks(i):
            b16 = k * SUB + i * L
            nt16 = nt_v[pl.ds(b16, L)]
            niv16 = niv_v[pl.ds(b16, L)]
            for r in range(L):
                rr = jnp.full((L,), r, dtype=jnp.int32)
                ntb = jnp.take_along_axis(nt16, rr, axis=0)
                nivb = jnp.take_along_axis(niv16, rr, axis=0)
                emb = plsc.load_gather(tab_v, [ntb * EMBED_DIM + lane])
                g = 2 * i + (r // 8)
                sl = r % 8
                buf[par, g, sl, pl.ds(0, L)] = emb
                buf[par, g, sl, pl.ds(L, L)] = nivb * wv + bv

        @pl.when(k > 0)
        def _():
            dma(k, par).wait()

        dma(k, par).start()
        return carry

    lax.fori_loop(0, nsub, chunk_body, 0)
    dma(0, 0).wait()


@jax.jit
def _sc_call(nt, niv, tab, wb):
    mesh = plsc.VectorSubcoreMesh(
        core_axis_name="c", subcore_axis_name="s",
        num_cores=NC, num_subcores=NS,
    )
    f = pl.kernel(
        _sc_body,
        out_type=jax.ShapeDtypeStruct((NG, 8, OUT_DIM), jnp.float32),
        mesh=mesh,
        compiler_params=pltpu.CompilerParams(needs_layout_passes=False),
        scratch_types=[
            pltpu.VMEM((ROWS_W,), jnp.int32),
            pltpu.VMEM((ROWS_W,), jnp.float32),
            pltpu.VMEM((3 * EMBED_DIM,), jnp.float32),
            pltpu.VMEM((2 * EMBED_DIM,), jnp.float32),
            pltpu.VMEM((2, GSUB, 8, OUT_DIM), jnp.float32),
            pltpu.SemaphoreType.DMA,
        ],
    )
    return f(nt, niv, tab, wb)


def kernel(node_type, num_inverted_predecessors, embed_table, W, b):
    nt = node_type.astype(jnp.int32)
    wb = jnp.concatenate([W.T, b[None, :]], axis=0)
    out_g = _sc_call(nt, num_inverted_predecessors,
                     embed_table.reshape(-1), wb.reshape(-1))
    return out_g.reshape(N, OUT_DIM)
